# baseline (speedup 1.0000x reference)
import functools

import jax
import jax.numpy as jnp
import numpy as np
from jax.experimental import pallas as pl
from jax.experimental.pallas import tpu as pltpu
from jax.experimental.pallas import tpu_sc as plsc

_OBJ_SCALE = 1.0
_NOOBJ_SCALE = 100.0
_IGNORE = 0.5
_ANCH = np.array([[10.0, 13.0], [16.0, 30.0], [33.0, 23.0]], dtype=np.float32)
_NA, _NB, _NH, _NW, _NGT = 3, 16, 64, 64, 256
_NC = _NA * (80 + 5)
_EPS = 1e-12
_TOTAL_CELLS = float(_NB * _NA * _NH * _NW)
_NCONF = _NB * _NA * _NH * _NW
_SC_W = 32
_PER_W = _NCONF // _SC_W


_NCELL = _NB * _NH * _NW
_CPW = _NCELL // _SC_W
_GSTART = (0, 88, 168)
_GOFF = tuple(85 * a + 4 - _GSTART[a] for a in range(_NA))


def _sc_pack_body(outV2_ref, confP_ref, bufs_ref, packed_ref, sem):
    wid = jax.lax.axis_index("s") * 2 + jax.lax.axis_index("c")
    base = wid * _CPW
    cps = [pltpu.make_async_copy(
        outV2_ref.at[pl.ds(base, _CPW), pl.ds(_GSTART[a], 8)],
        bufs_ref.at[a], sem) for a in range(_NA)]
    for cp in cps:
        cp.start()
    for cp in cps:
        cp.wait()

    lane = jax.lax.iota(jnp.int32, 16)

    def pick(j, carry):
        row = j * 16 + lane
        for a in range(_NA):
            v = plsc.load_gather(
                bufs_ref, [jnp.full((16,), a, jnp.int32), row,
                           jnp.full((16,), _GOFF[a], jnp.int32)])
            packed_ref[a, pl.ds(j * 16, 16)] = v
        return carry
    jax.lax.fori_loop(0, _CPW // 16, pick, 0)

    for a in range(_NA):
        pltpu.sync_copy(packed_ref.at[a],
                        confP_ref.at[pl.ds(a * _NCELL + base, _CPW)])


def _make_sc_pack():
    return pl.kernel(
        _sc_pack_body,
        out_type=jax.ShapeDtypeStruct((_NCONF,), jnp.float32),
        mesh=plsc.VectorSubcoreMesh(core_axis_name="c", subcore_axis_name="s"),
        compiler_params=pltpu.CompilerParams(use_tc_tiling_on_sc=False,
                                             needs_layout_passes=False),
        scratch_types=[
            pltpu.VMEM((_NA, _CPW, 8), jnp.float32),
            pltpu.VMEM((_NA, _CPW), jnp.float32),
            pltpu.SemaphoreType.DMA,
        ])


def _body(gr_ref, gc_ref, anch_ref, bat_ref, cell_ref, outV_ref, conf_ref,
          out_ref, gtv_ref, sem1):
    def issue(g, carry):
        b = bat_ref[g]
        cell8 = pl.multiple_of((cell_ref[g] // 8) * 8, 8)
        pltpu.make_async_copy(outV_ref.at[b, pl.ds(cell8, 8), :],
                              gtv_ref.at[g], sem1).start()
        return carry
    jax.lax.fori_loop(0, _NGT, issue, 0)

    gx_r = gr_ref[1:2, :] * _NW
    gy_r = gr_ref[2:3, :] * _NH
    gw_r = gr_ref[3:4, :] * _NW
    gh_r = gr_ref[4:5, :] * _NH
    gw_c = gc_ref[:, 3:4] * _NW
    gh_c = gc_ref[:, 4:5] * _NH

    def iou_ab(gw, gh, k):
        aw = anch_ref[0, k]
        ah = anch_ref[1, k]
        inter = jnp.minimum(gw, aw) * jnp.minimum(gh, ah)
        union = gw * gh + aw * ah - inter
        return inter / (union + 1e-16)

    iou_r = [iou_ab(gw_r, gh_r, k) for k in range(_NA)]
    iou_c = [iou_ab(gw_c, gh_c, k) for k in range(_NA)]

    def argmax3(i0, i1, i2):
        best = jnp.zeros_like(i0, dtype=jnp.int32)
        m = i0
        best = jnp.where(i1 > m, 1, best)
        m = jnp.maximum(m, i1)
        best = jnp.where(i2 > m, 2, best)
        return best

    best_r = argmax3(*iou_r)
    best_c = argmax3(*iou_c)
    excl_r = [(iou_r[k] > _IGNORE) | (best_r == k) for k in range(_NA)]
    excl_c = [(iou_c[k] > _IGNORE) | (best_c == k) for k in range(_NA)]

    gx_c = gc_ref[:, 1:2] * _NW
    gy_c = gc_ref[:, 2:3] * _NH
    cell_r = (gr_ref[0:1, :] * _NH + jnp.floor(gy_r)) * _NW + jnp.floor(gx_r)
    cell_c = (gc_ref[:, 0:1] * _NH + jnp.floor(gy_c)) * _NW + jnp.floor(gx_c)

    io_g = jax.lax.broadcasted_iota(jnp.int32, (_NGT, _NGT), 1)
    io_gp = jax.lax.broadcasted_iota(jnp.int32, (_NGT, _NGT), 0)
    later = io_gp > io_g
    same_cell = (cell_r == cell_c) & later
    same_best = best_r == best_c
    winner_obj = ~jnp.any(same_cell & same_best, axis=0, keepdims=True)
    winner_excl = [
        excl_r[k] & ~jnp.any(same_cell & excl_c[k], axis=0, keepdims=True)
        for k in range(_NA)]

    fobj = winner_obj.astype(jnp.float32)
    n_obj = jnp.maximum(jnp.sum(fobj), 1.0)
    n_excl = sum(jnp.sum(w.astype(jnp.float32)) for w in winner_excl)
    n_noobj = jnp.maximum(_TOTAL_CELLS - n_excl, 1.0)

    x = conf_ref[...]
    dens = jnp.sum(jnp.log(1.0 + jnp.exp(x)))

    def drain(g, carry):
        pltpu.make_async_copy(outV_ref.at[0, pl.ds(0, 8), :],
                              gtv_ref.at[0], sem1).wait()
        return carry
    jax.lax.fori_loop(0, _NGT, drain, 0)

    cellmod = (cell_c.astype(jnp.int32) % 8).reshape(_NGT, 1, 1)
    s_io = jax.lax.broadcasted_iota(jnp.int32, (_NGT, 8, 5), 1)
    smask = s_io == cellmod
    va = [jnp.sum(jnp.where(smask, gtv_ref[:, :, 85 * a:85 * a + 5], 0.0),
                  axis=1) for a in range(_NA)]
    gt15 = jnp.concatenate(va, axis=1).T

    sel = [(best_r == k).astype(jnp.float32) for k in range(_NA)]
    pv = []
    for c in range(5):
        pv.append(sum(sel[k] * gt15[5 * k + c:5 * k + c + 1, :]
                      for k in range(_NA)))
    conf_a = [gt15[5 * k + 4:5 * k + 5, :] for k in range(_NA)]

    saw_sel = sum(sel[k] * anch_ref[0, k] for k in range(_NA))
    sah_sel = sum(sel[k] * anch_ref[1, k] for k in range(_NA))
    tx = gx_r - jnp.floor(gx_r)
    ty = gy_r - jnp.floor(gy_r)
    tw = gw_r / saw_sel
    th = gh_r / sah_sel

    xs = jax.nn.sigmoid(pv[0])
    ys = jax.nn.sigmoid(pv[1])
    bbox = (xs - tx) ** 2 + (ys - ty) ** 2 \
        + (pv[2] - jnp.log(tw)) ** 2 + (pv[3] - jnp.log(th)) ** 2
    obj_bce = -jnp.log(jax.nn.sigmoid(pv[4]) + _EPS)
    sum_bbox = jnp.sum(bbox * fobj)
    sum_objbce = jnp.sum(obj_bce * fobj)
    corr = sum(
        jnp.sum(jnp.where(winner_excl[k],
                          -jnp.log(1.0 - jax.nn.sigmoid(conf_a[k]) + _EPS),
                          0.0))
        for k in range(_NA))

    total = (sum_bbox + _OBJ_SCALE * sum_objbce) / n_obj \
        + _NOOBJ_SCALE * (dens - corr) / n_noobj
    out_ref[0, 0] = total


def kernel(out, gts, size):
    outV = out.transpose(0, 2, 3, 1).reshape(_NB, _NH * _NW, _NC)
    stride_h = (size[0] // _NH).astype(jnp.float32)
    stride_w = (size[1] // _NW).astype(jnp.float32)
    saw = jnp.asarray(_ANCH[:, 0]) / stride_w
    sah = jnp.asarray(_ANCH[:, 1]) / stride_h
    anch = jnp.stack([saw, sah])
    gts_r = gts.T
    bat_i = gts[:, 0].astype(jnp.int32)
    cell_i = (gts[:, 2] * _NH).astype(jnp.int32) * _NW \
        + (gts[:, 1] * _NW).astype(jnp.int32)

    outV2 = outV.reshape(_NB * _NH * _NW, _NC)
    confP = _make_sc_pack()(outV2)
    confP2 = confP.reshape(_NCONF // 128, 128)

    total = pl.pallas_call(
        _body,
        in_specs=[
            pl.BlockSpec((5, _NGT), lambda: (0, 0)),
            pl.BlockSpec((_NGT, 5), lambda: (0, 0)),
            pl.BlockSpec(memory_space=pltpu.SMEM),
            pl.BlockSpec(memory_space=pltpu.SMEM),
            pl.BlockSpec(memory_space=pltpu.SMEM),
            pl.BlockSpec(memory_space=pltpu.MemorySpace.HBM),
            pl.BlockSpec((_NCONF // 128, 128), lambda: (0, 0)),
        ],
        out_specs=pl.BlockSpec(memory_space=pltpu.SMEM),
        out_shape=jax.ShapeDtypeStruct((1, 1), jnp.float32),
        scratch_shapes=[
            pltpu.VMEM((_NGT, 8, _NC), jnp.float32),
            pltpu.SemaphoreType.DMA,
        ],
    )(gts_r, gts, anch, bat_i, cell_i, outV, confP2)
    return total.reshape(())


# speedup vs baseline: 2.1016x; 2.1016x over previous
import functools

import jax
import jax.numpy as jnp
import numpy as np
from jax.experimental import pallas as pl
from jax.experimental.pallas import tpu as pltpu
from jax.experimental.pallas import tpu_sc as plsc

_OBJ_SCALE = 1.0
_NOOBJ_SCALE = 100.0
_IGNORE = 0.5
_ANCH = np.array([[10.0, 13.0], [16.0, 30.0], [33.0, 23.0]], dtype=np.float32)
_NA, _NB, _NH, _NW, _NGT = 3, 16, 64, 64, 256
_NC = _NA * (80 + 5)
_EPS = 1e-12
_TOTAL_CELLS = float(_NB * _NA * _NH * _NW)
_NCONF = _NB * _NA * _NH * _NW
_SC_W = 32
_PER_W = _NCONF // _SC_W


def _body(gr_ref, gc_ref, anch_ref, bat_ref, cell_ref, outV_ref, blk_ref,
          out_ref, gtv_ref, dens_ref, sem1):
    b = pl.program_id(0)
    t = pl.program_id(1)

    @pl.when(jnp.logical_and(b == 0, t == 0))
    def _init():
        dens_ref[0, 0] = 0.0

        def issue(g, carry):
            bb = bat_ref[g]
            cell8 = pl.multiple_of((cell_ref[g] // 8) * 8, 8)
            pltpu.make_async_copy(outV_ref.at[bb, pl.ds(cell8, 8), :],
                                  gtv_ref.at[g], sem1).start()
            return carry
        jax.lax.fori_loop(0, _NGT, issue, 0)

    x = blk_ref[0]
    lio = jax.lax.broadcasted_iota(jnp.int32, (_NH * _NW, 128), 1)
    val = jnp.log(1.0 + jnp.exp(x))
    c0 = jnp.where((lio == 4) | (lio == 89), val, 0.0)
    c1 = jnp.where(lio == 174 - 128, val, 0.0)
    dens_ref[0, 0] += jnp.sum(jnp.where(t == 0, c0, c1))

    @pl.when(jnp.logical_and(b == _NB - 1, t == 1))
    def _final():
        _combine(gr_ref, gc_ref, anch_ref, outV_ref, out_ref, gtv_ref,
                 dens_ref, sem1)


def _combine(gr_ref, gc_ref, anch_ref, outV_ref, out_ref, gtv_ref,
             dens_ref, sem1):

    gx_r = gr_ref[1:2, :] * _NW
    gy_r = gr_ref[2:3, :] * _NH
    gw_r = gr_ref[3:4, :] * _NW
    gh_r = gr_ref[4:5, :] * _NH
    gw_c = gc_ref[:, 3:4] * _NW
    gh_c = gc_ref[:, 4:5] * _NH

    def iou_ab(gw, gh, k):
        aw = anch_ref[0, k]
        ah = anch_ref[1, k]
        inter = jnp.minimum(gw, aw) * jnp.minimum(gh, ah)
        union = gw * gh + aw * ah - inter
        return inter / (union + 1e-16)

    iou_r = [iou_ab(gw_r, gh_r, k) for k in range(_NA)]
    iou_c = [iou_ab(gw_c, gh_c, k) for k in range(_NA)]

    def argmax3(i0, i1, i2):
        best = jnp.zeros_like(i0, dtype=jnp.int32)
        m = i0
        best = jnp.where(i1 > m, 1, best)
        m = jnp.maximum(m, i1)
        best = jnp.where(i2 > m, 2, best)
        return best

    best_r = argmax3(*iou_r)
    best_c = argmax3(*iou_c)
    excl_r = [(iou_r[k] > _IGNORE) | (best_r == k) for k in range(_NA)]
    excl_c = [(iou_c[k] > _IGNORE) | (best_c == k) for k in range(_NA)]

    gx_c = gc_ref[:, 1:2] * _NW
    gy_c = gc_ref[:, 2:3] * _NH
    cell_r = (gr_ref[0:1, :] * _NH + jnp.floor(gy_r)) * _NW + jnp.floor(gx_r)
    cell_c = (gc_ref[:, 0:1] * _NH + jnp.floor(gy_c)) * _NW + jnp.floor(gx_c)

    io_g = jax.lax.broadcasted_iota(jnp.int32, (_NGT, _NGT), 1)
    io_gp = jax.lax.broadcasted_iota(jnp.int32, (_NGT, _NGT), 0)
    later = io_gp > io_g
    same_cell = (cell_r == cell_c) & later
    same_best = best_r == best_c
    winner_obj = ~jnp.any(same_cell & same_best, axis=0, keepdims=True)
    winner_excl = [
        excl_r[k] & ~jnp.any(same_cell & excl_c[k], axis=0, keepdims=True)
        for k in range(_NA)]

    fobj = winner_obj.astype(jnp.float32)
    n_obj = jnp.maximum(jnp.sum(fobj), 1.0)
    n_excl = sum(jnp.sum(w.astype(jnp.float32)) for w in winner_excl)
    n_noobj = jnp.maximum(_TOTAL_CELLS - n_excl, 1.0)

    dens = dens_ref[0, 0]

    def drain(g, carry):
        pltpu.make_async_copy(outV_ref.at[0, pl.ds(0, 8), :],
                              gtv_ref.at[0], sem1).wait()
        return carry
    jax.lax.fori_loop(0, _NGT, drain, 0)

    cellmod = (cell_c.astype(jnp.int32) % 8).reshape(_NGT, 1, 1)
    s_io = jax.lax.broadcasted_iota(jnp.int32, (_NGT, 8, 5), 1)
    smask = s_io == cellmod
    va = [jnp.sum(jnp.where(smask, gtv_ref[:, :, 85 * a:85 * a + 5], 0.0),
                  axis=1) for a in range(_NA)]
    gt15 = jnp.concatenate(va, axis=1).T

    sel = [(best_r == k).astype(jnp.float32) for k in range(_NA)]
    pv = []
    for c in range(5):
        pv.append(sum(sel[k] * gt15[5 * k + c:5 * k + c + 1, :]
                      for k in range(_NA)))
    conf_a = [gt15[5 * k + 4:5 * k + 5, :] for k in range(_NA)]

    saw_sel = sum(sel[k] * anch_ref[0, k] for k in range(_NA))
    sah_sel = sum(sel[k] * anch_ref[1, k] for k in range(_NA))
    tx = gx_r - jnp.floor(gx_r)
    ty = gy_r - jnp.floor(gy_r)
    tw = gw_r / saw_sel
    th = gh_r / sah_sel

    xs = jax.nn.sigmoid(pv[0])
    ys = jax.nn.sigmoid(pv[1])
    bbox = (xs - tx) ** 2 + (ys - ty) ** 2 \
        + (pv[2] - jnp.log(tw)) ** 2 + (pv[3] - jnp.log(th)) ** 2
    obj_bce = -jnp.log(jax.nn.sigmoid(pv[4]) + _EPS)
    sum_bbox = jnp.sum(bbox * fobj)
    sum_objbce = jnp.sum(obj_bce * fobj)
    corr = sum(
        jnp.sum(jnp.where(winner_excl[k],
                          -jnp.log(1.0 - jax.nn.sigmoid(conf_a[k]) + _EPS),
                          0.0))
        for k in range(_NA))

    total = (sum_bbox + _OBJ_SCALE * sum_objbce) / n_obj \
        + _NOOBJ_SCALE * (dens - corr) / n_noobj
    out_ref[0, 0] = total


def kernel(out, gts, size):
    outV = out.transpose(0, 2, 3, 1).reshape(_NB, _NH * _NW, _NC)
    stride_h = (size[0] // _NH).astype(jnp.float32)
    stride_w = (size[1] // _NW).astype(jnp.float32)
    saw = jnp.asarray(_ANCH[:, 0]) / stride_w
    sah = jnp.asarray(_ANCH[:, 1]) / stride_h
    anch = jnp.stack([saw, sah])
    gts_r = gts.T
    bat_i = gts[:, 0].astype(jnp.int32)
    cell_i = (gts[:, 2] * _NH).astype(jnp.int32) * _NW \
        + (gts[:, 1] * _NW).astype(jnp.int32)

    total = pl.pallas_call(
        _body,
        grid=(_NB, 2),
        in_specs=[
            pl.BlockSpec((5, _NGT), lambda b, t: (0, 0)),
            pl.BlockSpec((_NGT, 5), lambda b, t: (0, 0)),
            pl.BlockSpec(memory_space=pltpu.SMEM),
            pl.BlockSpec(memory_space=pltpu.SMEM),
            pl.BlockSpec(memory_space=pltpu.SMEM),
            pl.BlockSpec(memory_space=pltpu.MemorySpace.HBM),
            pl.BlockSpec((1, _NH * _NW, 128), lambda b, t: (b, 0, t)),
        ],
        out_specs=pl.BlockSpec(memory_space=pltpu.SMEM),
        out_shape=jax.ShapeDtypeStruct((1, 1), jnp.float32),
        scratch_shapes=[
            pltpu.VMEM((_NGT, 8, _NC), jnp.float32),
            pltpu.SMEM((1, 1), jnp.float32),
            pltpu.SemaphoreType.DMA,
        ],
    )(gts_r, gts, anch, bat_i, cell_i, outV, outV)
    return total.reshape(())


# speedup vs baseline: 2.6371x; 1.2549x over previous
import functools

import jax
import jax.numpy as jnp
import numpy as np
from jax.experimental import pallas as pl
from jax.experimental.pallas import tpu as pltpu
from jax.experimental.pallas import tpu_sc as plsc

_OBJ_SCALE = 1.0
_NOOBJ_SCALE = 100.0
_IGNORE = 0.5
_ANCH = np.array([[10.0, 13.0], [16.0, 30.0], [33.0, 23.0]], dtype=np.float32)
_NA, _NB, _NH, _NW, _NGT = 3, 16, 64, 64, 256
_NC = _NA * (80 + 5)
_EPS = 1e-12
_TOTAL_CELLS = float(_NB * _NA * _NH * _NW)
_BB = 4


def _body(gr_ref, gc_ref, anch_ref, bat_ref, cell_ref, outV_ref, blk_ref,
          out_ref, gtv_ref, dens_ref, sem1):
    b = pl.program_id(0)
    t = pl.program_id(1)

    @pl.when(jnp.logical_and(b == 0, t == 0))
    def _init():
        dens_ref[0, 0] = 0.0

        def issue(g, carry):
            bb = bat_ref[g]
            cell8 = pl.multiple_of((cell_ref[g] // 8) * 8, 8)
            pltpu.make_async_copy(outV_ref.at[bb, pl.ds(cell8, 8), :],
                                  gtv_ref.at[g], sem1).start()
            return carry
        jax.lax.fori_loop(0, _NGT, issue, 0)

    x = blk_ref[...]
    lio = jax.lax.broadcasted_iota(jnp.int32, (_BB, _NH * _NW, 128), 2)
    val = jnp.log(1.0 + jnp.exp(x))
    c0 = jnp.where((lio == 4) | (lio == 89), val, 0.0)
    c1 = jnp.where(lio == 174 - 128, val, 0.0)
    dens_ref[0, 0] += jnp.sum(jnp.where(t == 0, c0, c1))

    @pl.when(jnp.logical_and(b == _NB // _BB - 1, t == 1))
    def _final():
        _combine(gr_ref, gc_ref, anch_ref, outV_ref, out_ref, gtv_ref,
                 dens_ref, sem1)


def _combine(gr_ref, gc_ref, anch_ref, outV_ref, out_ref, gtv_ref,
             dens_ref, sem1):

    gx_r = gr_ref[1:2, :] * _NW
    gy_r = gr_ref[2:3, :] * _NH
    gw_r = gr_ref[3:4, :] * _NW
    gh_r = gr_ref[4:5, :] * _NH
    gw_c = gc_ref[:, 3:4] * _NW
    gh_c = gc_ref[:, 4:5] * _NH

    def iou_ab(gw, gh, k):
        aw = anch_ref[0, k]
        ah = anch_ref[1, k]
        inter = jnp.minimum(gw, aw) * jnp.minimum(gh, ah)
        union = gw * gh + aw * ah - inter
        return inter / (union + 1e-16)

    iou_r = [iou_ab(gw_r, gh_r, k) for k in range(_NA)]
    iou_c = [iou_ab(gw_c, gh_c, k) for k in range(_NA)]

    def argmax3(i0, i1, i2):
        best = jnp.zeros_like(i0, dtype=jnp.int32)
        m = i0
        best = jnp.where(i1 > m, 1, best)
        m = jnp.maximum(m, i1)
        best = jnp.where(i2 > m, 2, best)
        return best

    best_r = argmax3(*iou_r)
    best_c = argmax3(*iou_c)
    excl_r = [(iou_r[k] > _IGNORE) | (best_r == k) for k in range(_NA)]
    excl_c = [(iou_c[k] > _IGNORE) | (best_c == k) for k in range(_NA)]

    gx_c = gc_ref[:, 1:2] * _NW
    gy_c = gc_ref[:, 2:3] * _NH
    cell_r = (gr_ref[0:1, :] * _NH + jnp.floor(gy_r)) * _NW + jnp.floor(gx_r)
    cell_c = (gc_ref[:, 0:1] * _NH + jnp.floor(gy_c)) * _NW + jnp.floor(gx_c)

    io_g = jax.lax.broadcasted_iota(jnp.int32, (_NGT, _NGT), 1)
    io_gp = jax.lax.broadcasted_iota(jnp.int32, (_NGT, _NGT), 0)
    later = io_gp > io_g
    same_cell = (cell_r == cell_c) & later
    same_best = best_r == best_c
    winner_obj = ~jnp.any(same_cell & same_best, axis=0, keepdims=True)
    winner_excl = [
        excl_r[k] & ~jnp.any(same_cell & excl_c[k], axis=0, keepdims=True)
        for k in range(_NA)]

    fobj = winner_obj.astype(jnp.float32)
    n_obj = jnp.maximum(jnp.sum(fobj), 1.0)
    n_excl = sum(jnp.sum(w.astype(jnp.float32)) for w in winner_excl)
    n_noobj = jnp.maximum(_TOTAL_CELLS - n_excl, 1.0)

    dens = dens_ref[0, 0]

    def drain(g, carry):
        pltpu.make_async_copy(outV_ref.at[0, pl.ds(0, 8), :],
                              gtv_ref.at[0], sem1).wait()
        return carry
    jax.lax.fori_loop(0, _NGT, drain, 0)

    cellmod = (cell_c.astype(jnp.int32) % 8).reshape(_NGT, 1, 1)
    s_io = jax.lax.broadcasted_iota(jnp.int32, (_NGT, 8, 5), 1)
    smask = s_io == cellmod
    va = [jnp.sum(jnp.where(smask, gtv_ref[:, :, 85 * a:85 * a + 5], 0.0),
                  axis=1) for a in range(_NA)]
    gt15 = jnp.concatenate(va, axis=1).T

    sel = [(best_r == k).astype(jnp.float32) for k in range(_NA)]
    pv = []
    for c in range(5):
        pv.append(sum(sel[k] * gt15[5 * k + c:5 * k + c + 1, :]
                      for k in range(_NA)))
    conf_a = [gt15[5 * k + 4:5 * k + 5, :] for k in range(_NA)]

    saw_sel = sum(sel[k] * anch_ref[0, k] for k in range(_NA))
    sah_sel = sum(sel[k] * anch_ref[1, k] for k in range(_NA))
    tx = gx_r - jnp.floor(gx_r)
    ty = gy_r - jnp.floor(gy_r)
    tw = gw_r / saw_sel
    th = gh_r / sah_sel

    xs = jax.nn.sigmoid(pv[0])
    ys = jax.nn.sigmoid(pv[1])
    bbox = (xs - tx) ** 2 + (ys - ty) ** 2 \
        + (pv[2] - jnp.log(tw)) ** 2 + (pv[3] - jnp.log(th)) ** 2
    obj_bce = -jnp.log(jax.nn.sigmoid(pv[4]) + _EPS)
    sum_bbox = jnp.sum(bbox * fobj)
    sum_objbce = jnp.sum(obj_bce * fobj)
    corr = sum(
        jnp.sum(jnp.where(winner_excl[k],
                          -jnp.log(1.0 - jax.nn.sigmoid(conf_a[k]) + _EPS),
                          0.0))
        for k in range(_NA))

    total = (sum_bbox + _OBJ_SCALE * sum_objbce) / n_obj \
        + _NOOBJ_SCALE * (dens - corr) / n_noobj
    out_ref[0, 0] = total


def kernel(out, gts, size):
    outV = out.transpose(0, 2, 3, 1).reshape(_NB, _NH * _NW, _NC)
    stride_h = (size[0] // _NH).astype(jnp.float32)
    stride_w = (size[1] // _NW).astype(jnp.float32)
    saw = jnp.asarray(_ANCH[:, 0]) / stride_w
    sah = jnp.asarray(_ANCH[:, 1]) / stride_h
    anch = jnp.stack([saw, sah])
    gts_r = gts.T
    bat_i = gts[:, 0].astype(jnp.int32)
    cell_i = (gts[:, 2] * _NH).astype(jnp.int32) * _NW \
        + (gts[:, 1] * _NW).astype(jnp.int32)

    total = pl.pallas_call(
        _body,
        grid=(_NB // _BB, 2),
        in_specs=[
            pl.BlockSpec((5, _NGT), lambda b, t: (0, 0)),
            pl.BlockSpec((_NGT, 5), lambda b, t: (0, 0)),
            pl.BlockSpec(memory_space=pltpu.SMEM),
            pl.BlockSpec(memory_space=pltpu.SMEM),
            pl.BlockSpec(memory_space=pltpu.SMEM),
            pl.BlockSpec(memory_space=pltpu.MemorySpace.HBM),
            pl.BlockSpec((_BB, _NH * _NW, 128), lambda b, t: (b, 0, t)),
        ],
        out_specs=pl.BlockSpec(memory_space=pltpu.SMEM),
        out_shape=jax.ShapeDtypeStruct((1, 1), jnp.float32),
        scratch_shapes=[
            pltpu.VMEM((_NGT, 8, _NC), jnp.float32),
            pltpu.SMEM((1, 1), jnp.float32),
            pltpu.SemaphoreType.DMA,
        ],
    )(gts_r, gts, anch, bat_i, cell_i, outV, outV)
    return total.reshape(())
